# baseline (speedup 1.0000x reference)
import functools

import jax
import jax.numpy as jnp
from jax import lax
from jax.experimental import pallas as pl
from jax.experimental.pallas import tpu as pltpu
from jax.experimental.pallas import tpu_sc as plsc

N = 8192
D = 2048
HALF = N // 2
NUM_SUBCORES = 16
ROWS = HALF // NUM_SUBCORES


def _body(x3, out):
    c = lax.axis_index("c")
    s = lax.axis_index("s")
    j0 = s * ROWS
    pltpu.sync_copy(
        x3.at[pl.ds(j0, ROWS), c],
        out.at[pl.ds(c * HALF + j0, ROWS)],
    )


@jax.jit
def _shuffle(x3):
    mesh = plsc.VectorSubcoreMesh(core_axis_name="c", subcore_axis_name="s")
    return pl.kernel(
        _body,
        out_type=jax.ShapeDtypeStruct((N, D), jnp.float32),
        mesh=mesh,
    )(x3)


def kernel(inputs):
    x3 = inputs.reshape(HALF, 2, D)
    return _shuffle(x3)


# speedup vs baseline: 14.8327x; 14.8327x over previous
import functools

import jax
import jax.numpy as jnp
from jax import lax
from jax.experimental import pallas as pl
from jax.experimental.pallas import tpu as pltpu
from jax.experimental.pallas import tpu_sc as plsc

N = 8192
D = 2048
HALF = N // 2
NUM_SUBCORES = 16
ROWS = HALF // NUM_SUBCORES
R = 8
C = ROWS // R


def _body(x3, out, buf, in_sems, out_sems):
    c = lax.axis_index("c")
    s = lax.axis_index("s")
    j0 = s * ROWS
    o0 = c * HALF + j0

    def start_in(k, slot):
        return pltpu.async_copy(
            x3.at[pl.ds(j0 + k * R, R), c], buf.at[slot], in_sems.at[slot]
        )

    def start_out(k, slot):
        return pltpu.async_copy(
            buf.at[slot], out.at[pl.ds(o0 + k * R, R)], out_sems.at[slot]
        )

    ins = [None] * C
    outs = [None] * C
    for k in range(C):
        slot = k % 2
        if k >= 2:
            outs[k - 2].wait()
        ins[k] = start_in(k, slot)
        if k >= 1:
            ins[k - 1].wait()
            outs[k - 1] = start_out(k - 1, (k - 1) % 2)
    ins[C - 1].wait()
    outs[C - 1] = start_out(C - 1, (C - 1) % 2)
    outs[C - 2].wait()
    outs[C - 1].wait()


@jax.jit
def _shuffle(x3):
    mesh = plsc.VectorSubcoreMesh(core_axis_name="c", subcore_axis_name="s")
    return pl.kernel(
        _body,
        out_type=jax.ShapeDtypeStruct((N, D), jnp.float32),
        mesh=mesh,
        scratch_types=[
            pltpu.VMEM((2, R, D), jnp.float32),
            pltpu.SemaphoreType.DMA((2,)),
            pltpu.SemaphoreType.DMA((2,)),
        ],
    )(x3)


def kernel(inputs):
    x3 = inputs.reshape(HALF, 2, D)
    return _shuffle(x3)


# speedup vs baseline: 31.3619x; 2.1144x over previous
import functools

import jax
import jax.numpy as jnp
from jax import lax
from jax.experimental import pallas as pl
from jax.experimental.pallas import tpu as pltpu
from jax.experimental.pallas import tpu_sc as plsc

N = 8192
D = 2048
HALF = N // 2
NUM_SUBCORES = 16
ROWS = HALF // NUM_SUBCORES
R = 16
C = ROWS // R


def _body(x, out, buf, in_sems, out_sems):
    h = lax.axis_index("c")
    t = lax.axis_index("s")
    o0 = h * HALF + t * ROWS
    lane = lax.iota(jnp.int32, 16)

    def start_in(k, slot):
        src_rows = (t * ROWS + k * R + lane) * 2 + h
        return pltpu.async_copy(x.at[src_rows], buf.at[slot], in_sems.at[slot])

    def start_out(k, slot):
        return pltpu.async_copy(
            buf.at[slot], out.at[pl.ds(o0 + k * R, R)], out_sems.at[slot]
        )

    ins = [None] * C
    outs = [None] * C
    for k in range(C):
        slot = k % 2
        if k >= 2:
            outs[k - 2].wait()
        ins[k] = start_in(k, slot)
        if k >= 1:
            ins[k - 1].wait()
            outs[k - 1] = start_out(k - 1, (k - 1) % 2)
    ins[C - 1].wait()
    outs[C - 1] = start_out(C - 1, (C - 1) % 2)
    outs[C - 2].wait()
    outs[C - 1].wait()


@jax.jit
def _shuffle(x):
    mesh = plsc.VectorSubcoreMesh(core_axis_name="c", subcore_axis_name="s")
    return pl.kernel(
        _body,
        out_type=jax.ShapeDtypeStruct((N, D), jnp.float32),
        mesh=mesh,
        scratch_types=[
            pltpu.VMEM((2, R, D), jnp.float32),
            pltpu.SemaphoreType.DMA((2,)),
            pltpu.SemaphoreType.DMA((2,)),
        ],
    )(x)


def kernel(inputs):
    return _shuffle(inputs)
